# baseline (speedup 1.0000x reference)
import functools

import jax
import jax.numpy as jnp
from jax.experimental import pallas as pl
from jax.experimental.pallas import tpu as pltpu

K = 5
EPS = 1e-8
N = 4096
D = 128
BCOL = 1024
NCHAIN = 4
NEG = -3.0e38


def _insert(tops, v):
    new = []
    for i, t in enumerate(tops):
        hi = jnp.maximum(t, v)
        if i + 1 < len(tops):
            v = jnp.minimum(t, v)
        new.append(hi)
    return new


def _merge(a, b, out_len):
    out = []
    for i in range(1, out_len + 1):
        terms = []
        if i <= len(a):
            terms.append(a[i - 1])
        if i <= len(b):
            terms.append(b[i - 1])
        for j in range(1, i):
            l = i - j
            if j <= len(a) and l <= len(b):
                terms.append(jnp.minimum(a[j - 1], b[l - 1]))
        m = terms[0]
        for t in terms[1:]:
            m = jnp.maximum(m, t)
        out.append(m)
    return out


def _knn_loss_kernel(s_ref, t_ref, out_ref, acc_ref, tbf_ref):
    i = pl.program_id(0)
    nsteps = pl.num_programs(0)

    @pl.when(i == 0)
    def _cast_teacher():
        tbf_ref[...] = t_ref[...].astype(jnp.bfloat16)

    s = s_ref[...].astype(jnp.bfloat16)

    def fresh(n):
        return [jnp.full((8, BCOL), jnp.bfloat16(NEG))] * n

    groups = [{"c5": fresh(K), "a2": fresh(2), "b2": fresh(2), "c1": fresh(1)}
              for _ in range(NCHAIN)]

    t = tbf_ref[...]
    x = jax.lax.dot_general(
        t, s, (((1,), (1,)), ((), ())),
        preferred_element_type=jnp.float32,
    ).astype(jnp.bfloat16)

    h = jnp.maximum(x[:N // 2, :], x[N // 2:, :])
    l = jnp.minimum(x[:N // 2, :], x[N // 2:, :])
    q = N // 4
    hh = jnp.maximum(h[:q, :], h[q:, :])
    hl = jnp.minimum(h[:q, :], h[q:, :])
    lh = jnp.maximum(l[:q, :], l[q:, :])
    ll = jnp.minimum(l[:q, :], l[q:, :])

    for g in range(q // 8):
        gr = groups[g % NCHAIN]
        gr["c5"] = _insert(gr["c5"], hh[g * 8:g * 8 + 8, :])
        gr["a2"] = _insert(gr["a2"], hl[g * 8:g * 8 + 8, :])
        gr["b2"] = _insert(gr["b2"], lh[g * 8:g * 8 + 8, :])
        gr["c1"] = _insert(gr["c1"], ll[g * 8:g * 8 + 8, :])

    chains = []
    for gr in groups:
        m = _merge(gr["a2"], gr["b2"], 4)
        m = _merge(m, gr["c1"], K)
        chains.append(_merge(gr["c5"], m, K))

    while len(chains) > 1:
        chains = [_merge(chains[i2], chains[i2 + 1], K)
                  for i2 in range(0, len(chains), 2)]
    tops = chains[0]

    for half in (4, 2, 1):
        aa = [t_[:half, :] for t_ in tops]
        bb = [t_[half:, :] for t_ in tops]
        tops = _merge(aa, bb, K)

    total = tops[0].astype(jnp.float32)
    for t_ in tops[1:]:
        total = total + t_.astype(jnp.float32)
    mean5 = total * jnp.float32(1.0 / K)
    partial = jnp.sum(jnp.log(mean5 + jnp.float32(EPS)))

    @pl.when(i == 0)
    def _init():
        acc_ref[0] = jnp.float32(0.0)

    acc_ref[0] = acc_ref[0] + partial

    @pl.when(i == nsteps - 1)
    def _fin():
        out_ref[0] = -acc_ref[0] * jnp.float32(1.0 / N)


@functools.partial(jax.jit, static_argnames=("interpret",))
def kernel(student_output, teacher_output, interpret=False):
    nsteps = N // BCOL
    out = pl.pallas_call(
        _knn_loss_kernel,
        grid=(nsteps,),
        in_specs=[
            pl.BlockSpec((BCOL, D), lambda i: (i, 0)),
            pl.BlockSpec((N, D), lambda i: (0, 0)),
        ],
        out_specs=pl.BlockSpec(memory_space=pltpu.SMEM),
        out_shape=jax.ShapeDtypeStruct((1,), jnp.float32),
        scratch_shapes=[
            pltpu.SMEM((1,), jnp.float32),
            pltpu.VMEM((N, D), jnp.bfloat16),
        ],
        interpret=interpret,
    )(student_output, teacher_output)
    return jnp.reshape(out, ())


# speedup vs baseline: 1.1174x; 1.1174x over previous
import functools

import jax
import jax.numpy as jnp
from jax.experimental import pallas as pl
from jax.experimental.pallas import tpu as pltpu

K = 5
EPS = 1e-8
N = 4096
D = 128
BCOL = 2048
NCHAIN = 2
NEG = -3.0e38


def _insert(tops, v):
    new = []
    for i, t in enumerate(tops):
        hi = jnp.maximum(t, v)
        if i + 1 < len(tops):
            v = jnp.minimum(t, v)
        new.append(hi)
    return new


def _merge(a, b, out_len):
    out = []
    for i in range(1, out_len + 1):
        terms = []
        if i <= len(a):
            terms.append(a[i - 1])
        if i <= len(b):
            terms.append(b[i - 1])
        for j in range(1, i):
            l = i - j
            if j <= len(a) and l <= len(b):
                terms.append(jnp.minimum(a[j - 1], b[l - 1]))
        m = terms[0]
        for t in terms[1:]:
            m = jnp.maximum(m, t)
        out.append(m)
    return out


def _knn_loss_kernel(s_ref, t_ref, out_ref, acc_ref, tbf_ref):
    i = pl.program_id(0)
    nsteps = pl.num_programs(0)

    @pl.when(i == 0)
    def _cast_teacher():
        tbf_ref[...] = t_ref[...].astype(jnp.bfloat16)

    s = s_ref[...].astype(jnp.bfloat16)

    def fresh(n):
        return [jnp.full((8, BCOL), jnp.bfloat16(NEG))] * n

    groups = [{"c5": fresh(K), "a2": fresh(2)} for _ in range(NCHAIN)]

    t = tbf_ref[...]
    x = jax.lax.dot_general(
        t, s, (((1,), (1,)), ((), ())),
        preferred_element_type=jnp.float32,
    ).astype(jnp.bfloat16)

    h = jnp.maximum(x[:N // 2, :], x[N // 2:, :])
    l = jnp.minimum(x[:N // 2, :], x[N // 2:, :])

    for g in range(N // 16):
        gr = groups[g % NCHAIN]
        gr["c5"] = _insert(gr["c5"], h[g * 8:g * 8 + 8, :])
        gr["a2"] = _insert(gr["a2"], l[g * 8:g * 8 + 8, :])

    chains = [_merge(gr["c5"], gr["a2"], K) for gr in groups]

    while len(chains) > 1:
        chains = [_merge(chains[i2], chains[i2 + 1], K)
                  for i2 in range(0, len(chains), 2)]
    tops = chains[0]

    for half in (4, 2, 1):
        aa = [t_[:half, :] for t_ in tops]
        bb = [t_[half:, :] for t_ in tops]
        tops = _merge(aa, bb, K)

    total = tops[0].astype(jnp.float32)
    for t_ in tops[1:]:
        total = total + t_.astype(jnp.float32)
    mean5 = total * jnp.float32(1.0 / K)
    partial = jnp.sum(jnp.log(mean5 + jnp.float32(EPS)))

    @pl.when(i == 0)
    def _init():
        acc_ref[0] = jnp.float32(0.0)

    acc_ref[0] = acc_ref[0] + partial

    @pl.when(i == nsteps - 1)
    def _fin():
        out_ref[0] = -acc_ref[0] * jnp.float32(1.0 / N)


@functools.partial(jax.jit, static_argnames=("interpret",))
def kernel(student_output, teacher_output, interpret=False):
    nsteps = N // BCOL
    out = pl.pallas_call(
        _knn_loss_kernel,
        grid=(nsteps,),
        in_specs=[
            pl.BlockSpec((BCOL, D), lambda i: (i, 0)),
            pl.BlockSpec((N, D), lambda i: (0, 0)),
        ],
        out_specs=pl.BlockSpec(memory_space=pltpu.SMEM),
        out_shape=jax.ShapeDtypeStruct((1,), jnp.float32),
        scratch_shapes=[
            pltpu.SMEM((1,), jnp.float32),
            pltpu.VMEM((N, D), jnp.bfloat16),
        ],
        interpret=interpret,
    )(student_output, teacher_output)
    return jnp.reshape(out, ())
